# baseline (speedup 1.0000x reference)
import functools

import jax
import jax.numpy as jnp
from jax import lax
from jax.experimental import pallas as pl
from jax.experimental.pallas import tpu as pltpu
from jax.experimental.pallas import tpu_sc as plsc

NUM_NODES = 1000000
EMB = 64
N_IDX = 819200

NC, NS = 2, 16
NW = NC * NS
ROWS_PER_W = N_IDX // NW
IDX_TILE = 128
CHUNK = 1024
K = CHUNK // IDX_TILE
N_CHUNKS = ROWS_PER_W // CHUNK

_mesh = plsc.VectorSubcoreMesh(core_axis_name="c", subcore_axis_name="s")


@functools.partial(
    pl.kernel,
    out_type=jax.ShapeDtypeStruct((N_IDX, EMB), jnp.float32),
    mesh=_mesh,
    scratch_types=[
        pltpu.VMEM((K, IDX_TILE), jnp.int32),
        pltpu.VMEM((CHUNK, EMB), jnp.float32),
        pltpu.SemaphoreType.DMA,
    ],
    compiler_params=pltpu.CompilerParams(use_tc_tiling_on_sc=False),
)
def _gather_sc(idx_hbm, table_hbm, out_hbm, idx_v, rows_v, gsem):
    wid = lax.axis_index("s") * NC + lax.axis_index("c")
    chunk0 = wid * N_CHUNKS

    def body(i, carry):
        c = chunk0 + i
        pltpu.sync_copy(idx_hbm.at[pl.ds(c * K, K)], idx_v)
        cps = [
            pltpu.async_copy(
                table_hbm.at[idx_v.at[j]],
                rows_v.at[pl.ds(j * IDX_TILE, IDX_TILE)],
                gsem,
            )
            for j in range(K)
        ]
        for cp in cps:
            cp.wait()
        pltpu.sync_copy(rows_v, out_hbm.at[pl.ds(c * CHUNK, CHUNK)])
        return carry

    lax.fori_loop(0, N_CHUNKS, body, 0)


def kernel(node_idx, node_embs):
    idx2d = node_idx.reshape(N_IDX // IDX_TILE, IDX_TILE)
    return _gather_sc(idx2d, node_embs)


# speedup vs baseline: 1.0196x; 1.0196x over previous
import functools

import jax
import jax.numpy as jnp
from jax import lax
from jax.experimental import pallas as pl
from jax.experimental.pallas import tpu as pltpu
from jax.experimental.pallas import tpu_sc as plsc

NUM_NODES = 1000000
EMB = 64
N_IDX = 819200

NC, NS = 2, 16
NW = NC * NS
ROWS_PER_W = N_IDX // NW
SLOT = 128
NSLOTS = ROWS_PER_W // SLOT
RING = 8
G = 4
NT = NSLOTS // RING

_mesh = plsc.VectorSubcoreMesh(core_axis_name="c", subcore_axis_name="s")


@functools.partial(
    pl.kernel,
    out_type=jax.ShapeDtypeStruct((N_IDX, EMB), jnp.float32),
    mesh=_mesh,
    scratch_types=[
        pltpu.VMEM((NSLOTS, SLOT), jnp.int32),
        pltpu.VMEM((RING, SLOT, EMB), jnp.float32),
        pltpu.SemaphoreType.DMA((RING,)),
        pltpu.SemaphoreType.DMA((RING,)),
    ],
    compiler_params=pltpu.CompilerParams(use_tc_tiling_on_sc=False),
)
def _gather_sc(idx_hbm, table_hbm, out_hbm, idx_v, rows_v, gsem, ssem):
    wid = lax.axis_index("s") * NC + lax.axis_index("c")
    irow0 = wid * NSLOTS
    row0 = wid * ROWS_PER_W

    pltpu.sync_copy(idx_hbm.at[pl.ds(irow0, NSLOTS)], idx_v)

    def fire_gather(s, b):
        pltpu.async_copy(table_hbm.at[idx_v.at[s]], rows_v.at[b], gsem.at[b])

    def wait_gather(s, b):
        pltpu.make_async_copy(
            table_hbm.at[idx_v.at[s]], rows_v.at[b], gsem.at[b]
        ).wait()

    def fire_store(s, b):
        pltpu.async_copy(
            rows_v.at[b], out_hbm.at[pl.ds(row0 + s * SLOT, SLOT)], ssem.at[b]
        )

    def wait_store(s, b):
        pltpu.make_async_copy(
            rows_v.at[b], out_hbm.at[pl.ds(row0 + s * SLOT, SLOT)], ssem.at[b]
        ).wait()

    def body(t, carry):
        for b in range(RING):
            s = t * RING + b
            @pl.when(t > 0)
            def _():
                wait_store(s - RING, b)

            fire_gather(s, b)

            bl = (b - G) % RING

            @pl.when(s >= G)
            def _():
                wait_gather(s - G, bl)
                fire_store(s - G, bl)

        return carry

    lax.fori_loop(0, NT, body, 0)

    for k in range(G):
        s = NSLOTS - G + k
        wait_gather(s, s % RING)
        fire_store(s, s % RING)
    for k in range(RING):
        s = NSLOTS - RING + k
        wait_store(s, s % RING)


def kernel(node_idx, node_embs):
    idx2d = node_idx.reshape(N_IDX // SLOT, SLOT)
    return _gather_sc(idx2d, node_embs)
